# baseline (speedup 1.0000x reference)
import functools

import jax
import jax.numpy as jnp
from jax.experimental import pallas as pl
from jax.experimental.pallas import tpu as pltpu

_KAC = 8
_L = 32
_F = _L + 2 * _KAC + 2
_KPAD = 128


def _nnue_kernel(w_ref, feat_ref, phase_ref, out_ref):
    feat = feat_ref[...]
    phase = phase_ref[...]
    w8 = w_ref[...]

    prods = jax.lax.dot_general(
        w8, feat,
        dimension_numbers=(((1,), (1,)), ((), ())),
        preferred_element_type=jnp.float32)

    mg = prods[0:1, :]
    eg = prods[1:2, :]
    ka0 = jnp.maximum(prods[2:3, :], 0.0)
    ka1 = jnp.maximum(prods[3:4, :], 0.0)
    pawn0 = prods[4:5, :]
    pawn1 = prods[5:6, :]

    king_attack = ka0 * ka0 - ka1 * ka1

    stronger_side_pawns = jnp.where(eg < 0.0, pawn1, pawn0)
    d = 8.0 - stronger_side_pawns
    eg_scale = (128.0 - d * d) * (1.0 / 128.0)

    a = eg * eg_scale
    b = mg + king_attack
    score = a + phase * (b - a)

    out_ref[...] = 0.5 * jnp.tanh(0.5 * score) + 0.5


def _pack_w8(w_mg, w_eg, w_ka):
    w8 = jnp.zeros((8, _KPAD), jnp.float32)
    w8 = w8.at[0, 0:_L].set(w_mg.reshape(_L).astype(jnp.float32))
    w8 = w8.at[1, 0:_L].set(w_eg.reshape(_L).astype(jnp.float32))
    w8 = w8.at[2, _L:_L + _KAC].set(w_ka.reshape(_KAC).astype(jnp.float32))
    w8 = w8.at[3, _L + _KAC:_L + 2 * _KAC].set(w_ka.reshape(_KAC).astype(jnp.float32))
    w8 = w8.at[4, _F - 2].set(1.0)
    w8 = w8.at[5, _F - 1].set(1.0)
    return w8.astype(jnp.bfloat16)


@functools.partial(jax.jit, static_argnames=("tb",))
def _forward(features, phase, w_mg, w_eg, w_ka, *, tb=8192):
    B, Fdim = features.shape
    assert Fdim == _F and B % tb == 0

    featb = jnp.pad(features, ((0, 0), (0, _KPAD - _F))).astype(jnp.bfloat16)
    phase_row = phase.astype(jnp.float32).reshape(1, B)
    w8 = _pack_w8(w_mg, w_eg, w_ka)

    nblk = B // tb

    out_row = pl.pallas_call(
        _nnue_kernel,
        out_shape=jax.ShapeDtypeStruct((1, B), jnp.float32),
        grid_spec=pltpu.PrefetchScalarGridSpec(
            num_scalar_prefetch=0,
            grid=(nblk,),
            in_specs=[
                pl.BlockSpec((8, _KPAD), lambda i: (0, 0)),
                pl.BlockSpec((tb, _KPAD), lambda i: (i, 0)),
                pl.BlockSpec((1, tb), lambda i: (0, i)),
            ],
            out_specs=pl.BlockSpec((1, tb), lambda i: (0, i)),
        ),
        compiler_params=pltpu.CompilerParams(
            dimension_semantics=("parallel",),
            vmem_limit_bytes=48 * 1024 * 1024),
    )(w8, featb, phase_row)

    return out_row.reshape(B, 1)


def kernel(features, phase, w_mg, w_eg, w_ka):
    return _forward(features, phase, w_mg, w_eg, w_ka)


# speedup vs baseline: 1.1577x; 1.1577x over previous
import functools

import jax
import jax.numpy as jnp
from jax.experimental import pallas as pl
from jax.experimental.pallas import tpu as pltpu

_KAC = 8
_L = 32
_F = _L + 2 * _KAC + 2


def _nnue_kernel(w_ref, feat_ref, phase_ref, out_ref):
    feat = feat_ref[...]
    phase = phase_ref[...]
    w8 = w_ref[...]

    prods = jax.lax.dot_general(
        w8, feat,
        dimension_numbers=(((1,), (1,)), ((), ())),
        preferred_element_type=jnp.float32)

    mg = prods[0:1, :]
    eg = prods[1:2, :]
    ka0 = jnp.maximum(prods[2:3, :], 0.0)
    ka1 = jnp.maximum(prods[3:4, :], 0.0)
    pawn0 = prods[4:5, :]
    pawn1 = prods[5:6, :]

    king_attack = ka0 * ka0 - ka1 * ka1

    stronger_side_pawns = jnp.where(eg < 0.0, pawn1, pawn0)
    d = 8.0 - stronger_side_pawns
    eg_scale = (128.0 - d * d) * (1.0 / 128.0)

    a = eg * eg_scale
    b = mg + king_attack
    score = a + phase * (b - a)

    out_ref[...] = 0.5 * jnp.tanh(0.5 * score) + 0.5


def _pack_w8(w_mg, w_eg, w_ka):
    w8 = jnp.zeros((8, _F), jnp.float32)
    w8 = w8.at[0, 0:_L].set(w_mg.reshape(_L).astype(jnp.float32))
    w8 = w8.at[1, 0:_L].set(w_eg.reshape(_L).astype(jnp.float32))
    w8 = w8.at[2, _L:_L + _KAC].set(w_ka.reshape(_KAC).astype(jnp.float32))
    w8 = w8.at[3, _L + _KAC:_L + 2 * _KAC].set(w_ka.reshape(_KAC).astype(jnp.float32))
    w8 = w8.at[4, _F - 2].set(1.0)
    w8 = w8.at[5, _F - 1].set(1.0)
    return w8


@functools.partial(jax.jit, static_argnames=("tb",))
def _forward(features, phase, w_mg, w_eg, w_ka, *, tb=32768):
    B, Fdim = features.shape
    assert Fdim == _F

    feats = features.astype(jnp.float32)
    phase_row = phase.astype(jnp.float32).reshape(1, B)
    w8 = _pack_w8(w_mg, w_eg, w_ka)

    tb_eff = min(tb, B)
    nblk = pl.cdiv(B, tb_eff)

    out_row = pl.pallas_call(
        _nnue_kernel,
        out_shape=jax.ShapeDtypeStruct((1, B), jnp.float32),
        grid_spec=pltpu.PrefetchScalarGridSpec(
            num_scalar_prefetch=0,
            grid=(nblk,),
            in_specs=[
                pl.BlockSpec((8, _F), lambda i: (0, 0)),
                pl.BlockSpec((tb_eff, _F), lambda i: (i, 0)),
                pl.BlockSpec((1, tb_eff), lambda i: (0, i)),
            ],
            out_specs=pl.BlockSpec((1, tb_eff), lambda i: (0, i)),
        ),
        compiler_params=pltpu.CompilerParams(
            dimension_semantics=("parallel",),
            vmem_limit_bytes=48 * 1024 * 1024),
    )(w8, feats, phase_row)

    return out_row.reshape(B, 1)


def kernel(features, phase, w_mg, w_eg, w_ka):
    return _forward(features, phase, w_mg, w_eg, w_ka)
